# baseline (speedup 1.0000x reference)
import functools

import jax
import jax.numpy as jnp
from jax import lax
from jax.experimental import pallas as pl
from jax.experimental.pallas import tpu as pltpu
from jax.experimental.pallas import tpu_sc as plsc

N = 10000
H = 8
C = 16
D = 128
NPAD = 10112
NB = 8
BLK = NPAD // NB
NC = 2
NS = 16
CH = 112
NCHUNK = 90
EPT = CH * NCHUNK
EPAD = NC * NS * EPT
NSC = 10016
RPTS = NSC // NS


def _prep_body(x_ref, w_ref, as_ref, ad_ref, h_ref, at_ref, dt_ref, m_ref):
    i = pl.program_id(0)
    h = jnp.dot(x_ref[...], w_ref[...], preferred_element_type=jnp.float32)
    h_ref[...] = h
    a_s = jnp.dot(h, as_ref[...], preferred_element_type=jnp.float32)
    a_d = jnp.dot(h, ad_ref[...], preferred_element_type=jnp.float32)
    at_ref[...] = a_s
    dt_ref[...] = a_d
    ms = jnp.max(a_s, axis=0, keepdims=True)
    md = jnp.max(a_d, axis=0, keepdims=True)

    @pl.when(i == 0)
    def _():
        m_ref[...] = jnp.zeros((8, 16), jnp.float32)
        m_ref[0:1, :] = ms
        m_ref[1:2, :] = md

    @pl.when(i > 0)
    def _():
        m_ref[0:1, :] = jnp.maximum(m_ref[0:1, :], ms)
        m_ref[1:2, :] = jnp.maximum(m_ref[1:2, :], md)

    @pl.when(i == NB - 1)
    def _():
        m_ref[2:3, :] = jnp.maximum(m_ref[0:1, :] + m_ref[1:2, :], 0.0)


_prep = pl.pallas_call(
    _prep_body,
    grid=(NB,),
    in_specs=[
        pl.BlockSpec((BLK, D), lambda i: (i, 0)),
        pl.BlockSpec((D, D), lambda i: (0, 0)),
        pl.BlockSpec((D, 16), lambda i: (0, 0)),
        pl.BlockSpec((D, 16), lambda i: (0, 0)),
    ],
    out_specs=[
        pl.BlockSpec((BLK, D), lambda i: (i, 0)),
        pl.BlockSpec((BLK, 16), lambda i: (i, 0)),
        pl.BlockSpec((BLK, 16), lambda i: (i, 0)),
        pl.BlockSpec((8, 16), lambda i: (0, 0)),
    ],
    out_shape=[
        jax.ShapeDtypeStruct((NPAD, D), jnp.float32),
        jax.ShapeDtypeStruct((NPAD, 16), jnp.float32),
        jax.ShapeDtypeStruct((NPAD, 16), jnp.float32),
        jax.ShapeDtypeStruct((8, 16), jnp.float32),
    ],
)


def _edge_body(h_hbm, as_hbm, ad_hbm, m_hbm, sd_hbm, zacc_hbm,
               zden_hbm, acc_out, den_out,
               sdx0, sdx1, hbuf0, hbuf1, asb0, asb1, adb0, adb1,
               pbuf0, pbuf1, mbuf, acc_sh, den_sh,
               semh0, semh1, sema0, sema1, semd0, semd1,
               semp0, semp1, semm0, semm1):
    c = lax.axis_index("c")
    s = lax.axis_index("s")
    w = c * NS + s
    sdx = (sdx0, sdx1)
    hbuf = (hbuf0, hbuf1)
    asb = (asb0, asb1)
    adb = (adb0, adb1)
    pbuf = (pbuf0, pbuf1)
    semh = (semh0, semh1)
    sema = (sema0, sema1)
    semd = (semd0, semd1)
    semp = (semp0, semp1)
    semm = (semm0, semm1)

    pltpu.sync_copy(zacc_hbm.at[pl.ds(s * RPTS, RPTS)],
                    acc_sh.at[pl.ds(s * RPTS, RPTS)])
    pltpu.sync_copy(zden_hbm.at[pl.ds(s * RPTS, RPTS)],
                    den_sh.at[pl.ds(s * RPTS, RPTS)])
    pltpu.sync_copy(m_hbm, mbuf)
    plsc.subcore_barrier()

    def fire(ch, b):
        pltpu.sync_copy(sd_hbm.at[w].at[ch], sdx[b])
        sidx = sdx[b].at[0]
        didx = sdx[b].at[1]
        pltpu.async_copy(h_hbm.at[sidx], hbuf[b], semh[b])
        pltpu.async_copy(as_hbm.at[sidx], asb[b], sema[b])
        pltpu.async_copy(ad_hbm.at[didx], adb[b], semd[b])

    def wait_gathers(b):
        sidx = sdx[b].at[0]
        didx = sdx[b].at[1]
        pltpu.make_async_copy(h_hbm.at[sidx], hbuf[b], semh[b]).wait()
        pltpu.make_async_copy(as_hbm.at[sidx], asb[b], sema[b]).wait()
        pltpu.make_async_copy(ad_hbm.at[didx], adb[b], semd[b]).wait()

    def scatter(b):
        didx = sdx[b].at[1]
        pltpu.async_copy(pbuf[b], den_sh.at[didx], semp[b], add=True)
        pltpu.async_copy(hbuf[b], acc_sh.at[didx], semm[b], add=True)

    def wait_scatters(b):
        didx = sdx[b].at[1]
        pltpu.make_async_copy(pbuf[b], den_sh.at[didx], semp[b]).wait()
        pltpu.make_async_copy(hbuf[b], acc_sh.at[didx], semm[b]).wait()

    def compute(b):
        mreg = mbuf[...]
        hb = hbuf[b]
        ab = asb[b]
        db = adb[b]
        pb = pbuf[b]

        @plsc.parallel_loop(0, CH, unroll=4)
        def _(e):
            a = ab[e] + db[e]
            a = jnp.where(a > 0.0, a, 0.2 * a)
            p = jnp.exp(a - mreg)
            pb[e] = p
            for hh in range(H):
                hb[e, pl.ds(hh * C, C)] = hb[e, pl.ds(hh * C, C)] * p[hh]

    fire(0, 0)

    def pair_body(pp, carry):
        ch0 = 2 * pp

        wait_gathers(0)

        @pl.when(pp > 0)
        def _():
            wait_scatters(1)

        fire(ch0 + 1, 1)
        compute(0)
        scatter(0)

        wait_gathers(1)
        wait_scatters(0)

        @pl.when(pp < NCHUNK // 2 - 1)
        def _():
            fire(ch0 + 2, 0)

        compute(1)
        scatter(1)
        return carry

    lax.fori_loop(0, NCHUNK // 2, pair_body, 0)
    wait_scatters(1)
    plsc.subcore_barrier()
    pltpu.sync_copy(acc_sh.at[pl.ds(s * RPTS, RPTS)],
                    acc_out.at[c].at[pl.ds(s * RPTS, RPTS)])
    pltpu.sync_copy(den_sh.at[pl.ds(s * RPTS, RPTS)],
                    den_out.at[c].at[pl.ds(s * RPTS, RPTS)])


_edge = pl.kernel(
    _edge_body,
    out_type=[
        jax.ShapeDtypeStruct((NC, NPAD, D), jnp.float32),
        jax.ShapeDtypeStruct((NC, NPAD, 16), jnp.float32),
    ],
    mesh=plsc.VectorSubcoreMesh(core_axis_name="c", subcore_axis_name="s"),
    scratch_types=[
        pltpu.VMEM((2, CH), jnp.int32),
        pltpu.VMEM((2, CH), jnp.int32),
        pltpu.VMEM((CH, D), jnp.float32),
        pltpu.VMEM((CH, D), jnp.float32),
        pltpu.VMEM((CH, 16), jnp.float32),
        pltpu.VMEM((CH, 16), jnp.float32),
        pltpu.VMEM((CH, 16), jnp.float32),
        pltpu.VMEM((CH, 16), jnp.float32),
        pltpu.VMEM((CH, 16), jnp.float32),
        pltpu.VMEM((CH, 16), jnp.float32),
        pltpu.VMEM((16,), jnp.float32),
        pltpu.VMEM_SHARED((NSC, D), jnp.float32),
        pltpu.VMEM_SHARED((NSC, 16), jnp.float32),
    ] + [pltpu.SemaphoreType.DMA] * 10,
    compiler_params=pltpu.CompilerParams(use_tc_tiling_on_sc=False),
)


def _final_body(acc_ref, den_ref, h_ref, as_ref, ad_ref, m_ref, ex_ref,
                bg_ref, w2_ref, b2_ref, out_ref):
    a = as_ref[...] + ad_ref[...]
    a = jnp.where(a > 0.0, a, 0.2 * a)
    ps = jnp.exp(a - m_ref[2:3, :])
    accs = acc_ref[...]
    dens = den_ref[...]
    den = dens[0] + dens[1] + ps
    pex = jnp.dot(ps, ex_ref[...], preferred_element_type=jnp.float32)
    denx = jnp.dot(den, ex_ref[...], preferred_element_type=jnp.float32)
    acc = accs[0] + accs[1] + pex * h_ref[...]
    gat = jnp.maximum(acc / (denx + 1e-16) + bg_ref[0:1, :], 0.0)
    out_ref[...] = (jnp.dot(gat, w2_ref[...], preferred_element_type=jnp.float32)
                    + b2_ref[0:1, :])


_final = pl.pallas_call(
    _final_body,
    grid=(NB,),
    in_specs=[
        pl.BlockSpec((NC, BLK, D), lambda i: (0, i, 0)),
        pl.BlockSpec((NC, BLK, 16), lambda i: (0, i, 0)),
        pl.BlockSpec((BLK, D), lambda i: (i, 0)),
        pl.BlockSpec((BLK, 16), lambda i: (i, 0)),
        pl.BlockSpec((BLK, 16), lambda i: (i, 0)),
        pl.BlockSpec((8, 16), lambda i: (0, 0)),
        pl.BlockSpec((16, D), lambda i: (0, 0)),
        pl.BlockSpec((8, D), lambda i: (0, 0)),
        pl.BlockSpec((D, D), lambda i: (0, 0)),
        pl.BlockSpec((8, D), lambda i: (0, 0)),
    ],
    out_specs=pl.BlockSpec((BLK, D), lambda i: (i, 0)),
    out_shape=jax.ShapeDtypeStruct((NPAD, D), jnp.float32),
)


def kernel(x, edge_index, W, att_src, att_dst, bias_gat, W2, b2):
    f32 = jnp.float32
    x_pad = jnp.zeros((NPAD, D), f32).at[:N].set(x.astype(f32))

    hc = jnp.arange(D)
    head = hc // C
    ASmat = jnp.zeros((D, 16), f32).at[hc, head].set(att_src.reshape(D))
    ADmat = jnp.zeros((D, 16), f32).at[hc, head].set(att_dst.reshape(D))
    EXPAND = jnp.zeros((16, D), f32).at[head, hc].set(1.0)

    E = edge_index.shape[1]
    pad = jnp.full((2, EPAD - E), N, jnp.int32)
    sd = jnp.concatenate([edge_index.astype(jnp.int32), pad], axis=1)
    sd = sd.reshape(2, NC * NS, NCHUNK, CH).transpose(1, 2, 0, 3)

    h, asrc_tab, adst_tab, m8 = _prep(x_pad, W.astype(f32), ASmat, ADmat)

    zacc = jnp.zeros((NPAD, D), f32)
    zden = jnp.zeros((NPAD, 16), f32)
    mvec = m8[2]
    acc_p, den_p = _edge(h, asrc_tab, adst_tab, mvec, sd, zacc, zden)

    bg2 = jnp.broadcast_to(bias_gat.astype(f32).reshape(1, D), (8, D))
    b22 = jnp.broadcast_to(b2.astype(f32).reshape(1, D), (8, D))
    out = _final(acc_p, den_p, h, asrc_tab, adst_tab, m8, EXPAND, bg2,
                 W2.astype(f32), b22)
    return out[:N]


# speedup vs baseline: 1.0503x; 1.0503x over previous
import functools

import jax
import jax.numpy as jnp
from jax import lax
from jax.experimental import pallas as pl
from jax.experimental.pallas import tpu as pltpu
from jax.experimental.pallas import tpu_sc as plsc

N = 10000
H = 8
C = 16
D = 128
NB = 10
BLK = N // NB
NC = 2
NS = 16
CH = 112
NCHUNK = 90
EPT = CH * NCHUNK
EPAD = NC * NS * EPT
NSC = 10016
RPTS = NSC // NS


def _prep_body(x_ref, w_ref, as_ref, ad_ref, h_ref, at_ref, dt_ref, m_ref):
    i = pl.program_id(0)
    h = jnp.dot(x_ref[...], w_ref[...], preferred_element_type=jnp.float32)
    h_ref[...] = h
    a_s = jnp.dot(h, as_ref[...], preferred_element_type=jnp.float32)
    a_d = jnp.dot(h, ad_ref[...], preferred_element_type=jnp.float32)
    at_ref[...] = a_s
    dt_ref[...] = a_d
    ms = jnp.max(a_s, axis=0, keepdims=True)
    md = jnp.max(a_d, axis=0, keepdims=True)

    @pl.when(i == 0)
    def _():
        m_ref[...] = jnp.zeros((8, 16), jnp.float32)
        m_ref[0:1, :] = ms
        m_ref[1:2, :] = md

    @pl.when(i > 0)
    def _():
        m_ref[0:1, :] = jnp.maximum(m_ref[0:1, :], ms)
        m_ref[1:2, :] = jnp.maximum(m_ref[1:2, :], md)

    @pl.when(i == NB - 1)
    def _():
        m_ref[2:3, :] = jnp.maximum(m_ref[0:1, :] + m_ref[1:2, :], 0.0)


_prep = pl.pallas_call(
    _prep_body,
    grid=(NB,),
    in_specs=[
        pl.BlockSpec((BLK, D), lambda i: (i, 0)),
        pl.BlockSpec((D, D), lambda i: (0, 0)),
        pl.BlockSpec((D, 16), lambda i: (0, 0)),
        pl.BlockSpec((D, 16), lambda i: (0, 0)),
    ],
    out_specs=[
        pl.BlockSpec((BLK, D), lambda i: (i, 0)),
        pl.BlockSpec((BLK, 16), lambda i: (i, 0)),
        pl.BlockSpec((BLK, 16), lambda i: (i, 0)),
        pl.BlockSpec((8, 16), lambda i: (0, 0)),
    ],
    out_shape=[
        jax.ShapeDtypeStruct((N, D), jnp.float32),
        jax.ShapeDtypeStruct((N, 16), jnp.float32),
        jax.ShapeDtypeStruct((N, 16), jnp.float32),
        jax.ShapeDtypeStruct((8, 16), jnp.float32),
    ],
)


def _edge_body(h_hbm, as_hbm, ad_hbm, m_hbm, sd_hbm, zacc_hbm,
               zden_hbm, acc_out, den_out,
               sdx0, sdx1, hbuf0, hbuf1, asb0, asb1, adb0, adb1,
               pbuf0, pbuf1, mbuf, acc_sh, den_sh,
               semh0, semh1, sema0, sema1, semd0, semd1,
               semp0, semp1, semm0, semm1):
    c = lax.axis_index("c")
    s = lax.axis_index("s")
    w = c * NS + s
    sdx = (sdx0, sdx1)
    hbuf = (hbuf0, hbuf1)
    asb = (asb0, asb1)
    adb = (adb0, adb1)
    pbuf = (pbuf0, pbuf1)
    semh = (semh0, semh1)
    sema = (sema0, sema1)
    semd = (semd0, semd1)
    semp = (semp0, semp1)
    semm = (semm0, semm1)

    pltpu.sync_copy(zacc_hbm.at[pl.ds(s * RPTS, RPTS)],
                    acc_sh.at[pl.ds(s * RPTS, RPTS)])
    pltpu.sync_copy(zden_hbm.at[pl.ds(s * RPTS, RPTS)],
                    den_sh.at[pl.ds(s * RPTS, RPTS)])
    pltpu.sync_copy(m_hbm, mbuf)
    plsc.subcore_barrier()

    def fire(ch, b):
        pltpu.sync_copy(sd_hbm.at[w].at[ch], sdx[b])
        sidx = sdx[b].at[0]
        didx = sdx[b].at[1]
        pltpu.async_copy(h_hbm.at[sidx], hbuf[b], semh[b])
        pltpu.async_copy(as_hbm.at[sidx], asb[b], sema[b])
        pltpu.async_copy(ad_hbm.at[didx], adb[b], semd[b])

    def wait_gathers(b):
        sidx = sdx[b].at[0]
        didx = sdx[b].at[1]
        pltpu.make_async_copy(h_hbm.at[sidx], hbuf[b], semh[b]).wait()
        pltpu.make_async_copy(as_hbm.at[sidx], asb[b], sema[b]).wait()
        pltpu.make_async_copy(ad_hbm.at[didx], adb[b], semd[b]).wait()

    def scatter(b):
        didx = sdx[b].at[1]
        pltpu.async_copy(pbuf[b], den_sh.at[didx], semp[b], add=True)
        pltpu.async_copy(hbuf[b], acc_sh.at[didx], semm[b], add=True)

    def wait_scatters(b):
        didx = sdx[b].at[1]
        pltpu.make_async_copy(pbuf[b], den_sh.at[didx], semp[b]).wait()
        pltpu.make_async_copy(hbuf[b], acc_sh.at[didx], semm[b]).wait()

    def compute(b):
        mreg = mbuf[...]
        hb = hbuf[b]
        ab = asb[b]
        db = adb[b]
        pb = pbuf[b]

        @plsc.parallel_loop(0, CH, unroll=4)
        def _(e):
            a = ab[e] + db[e]
            a = jnp.where(a > 0.0, a, 0.2 * a)
            p = jnp.exp(a - mreg)
            pb[e] = p
            for hh in range(H):
                hb[e, pl.ds(hh * C, C)] = hb[e, pl.ds(hh * C, C)] * p[hh]

    fire(0, 0)

    def pair_body(pp, carry):
        ch0 = 2 * pp

        wait_gathers(0)

        @pl.when(pp > 0)
        def _():
            wait_scatters(1)

        fire(ch0 + 1, 1)
        compute(0)
        scatter(0)

        wait_gathers(1)
        wait_scatters(0)

        @pl.when(pp < NCHUNK // 2 - 1)
        def _():
            fire(ch0 + 2, 0)

        compute(1)
        scatter(1)
        return carry

    lax.fori_loop(0, NCHUNK // 2, pair_body, 0)
    wait_scatters(1)
    plsc.subcore_barrier()
    pltpu.sync_copy(acc_sh.at[pl.ds(s * RPTS, RPTS)],
                    acc_out.at[c].at[pl.ds(s * RPTS, RPTS)])
    pltpu.sync_copy(den_sh.at[pl.ds(s * RPTS, RPTS)],
                    den_out.at[c].at[pl.ds(s * RPTS, RPTS)])


_edge = pl.kernel(
    _edge_body,
    out_type=[
        jax.ShapeDtypeStruct((NC, NSC, D), jnp.float32),
        jax.ShapeDtypeStruct((NC, NSC, 16), jnp.float32),
    ],
    mesh=plsc.VectorSubcoreMesh(core_axis_name="c", subcore_axis_name="s"),
    scratch_types=[
        pltpu.VMEM((2, CH), jnp.int32),
        pltpu.VMEM((2, CH), jnp.int32),
        pltpu.VMEM((CH, D), jnp.float32),
        pltpu.VMEM((CH, D), jnp.float32),
        pltpu.VMEM((CH, 16), jnp.float32),
        pltpu.VMEM((CH, 16), jnp.float32),
        pltpu.VMEM((CH, 16), jnp.float32),
        pltpu.VMEM((CH, 16), jnp.float32),
        pltpu.VMEM((CH, 16), jnp.float32),
        pltpu.VMEM((CH, 16), jnp.float32),
        pltpu.VMEM((16,), jnp.float32),
        pltpu.VMEM_SHARED((NSC, D), jnp.float32),
        pltpu.VMEM_SHARED((NSC, 16), jnp.float32),
    ] + [pltpu.SemaphoreType.DMA] * 10,
    compiler_params=pltpu.CompilerParams(use_tc_tiling_on_sc=False),
)


def _final_body(acc_ref, den_ref, h_ref, as_ref, ad_ref, m_ref, ex_ref,
                bg_ref, w2_ref, b2_ref, out_ref):
    a = as_ref[...] + ad_ref[...]
    a = jnp.where(a > 0.0, a, 0.2 * a)
    ps = jnp.exp(a - m_ref[2:3, :])
    accs = acc_ref[...]
    dens = den_ref[...]
    den = dens[0] + dens[1] + ps
    pex = jnp.dot(ps, ex_ref[...], preferred_element_type=jnp.float32)
    denx = jnp.dot(den, ex_ref[...], preferred_element_type=jnp.float32)
    acc = accs[0] + accs[1] + pex * h_ref[...]
    gat = jnp.maximum(acc / (denx + 1e-16) + bg_ref[0:1, :], 0.0)
    out_ref[...] = (jnp.dot(gat, w2_ref[...], preferred_element_type=jnp.float32)
                    + b2_ref[0:1, :])


_final = pl.pallas_call(
    _final_body,
    grid=(NB,),
    in_specs=[
        pl.BlockSpec((NC, BLK, D), lambda i: (0, i, 0)),
        pl.BlockSpec((NC, BLK, 16), lambda i: (0, i, 0)),
        pl.BlockSpec((BLK, D), lambda i: (i, 0)),
        pl.BlockSpec((BLK, 16), lambda i: (i, 0)),
        pl.BlockSpec((BLK, 16), lambda i: (i, 0)),
        pl.BlockSpec((8, 16), lambda i: (0, 0)),
        pl.BlockSpec((16, D), lambda i: (0, 0)),
        pl.BlockSpec((8, D), lambda i: (0, 0)),
        pl.BlockSpec((D, D), lambda i: (0, 0)),
        pl.BlockSpec((8, D), lambda i: (0, 0)),
    ],
    out_specs=pl.BlockSpec((BLK, D), lambda i: (i, 0)),
    out_shape=jax.ShapeDtypeStruct((N, D), jnp.float32),
)


def kernel(x, edge_index, W, att_src, att_dst, bias_gat, W2, b2):
    f32 = jnp.float32

    hc = jnp.arange(D)
    head = hc // C
    ASmat = jnp.zeros((D, 16), f32).at[hc, head].set(att_src.reshape(D))
    ADmat = jnp.zeros((D, 16), f32).at[hc, head].set(att_dst.reshape(D))
    EXPAND = jnp.zeros((16, D), f32).at[head, hc].set(1.0)

    E = edge_index.shape[1]
    pads = jnp.zeros((1, EPAD - E), jnp.int32)
    padd = jnp.full((1, EPAD - E), N + 8, jnp.int32)
    sd = jnp.concatenate(
        [edge_index.astype(jnp.int32), jnp.concatenate([pads, padd], axis=0)],
        axis=1)
    sd = sd.reshape(2, NC * NS, NCHUNK, CH).transpose(1, 2, 0, 3)

    h, asrc_tab, adst_tab, m8 = _prep(x.astype(f32), W.astype(f32), ASmat,
                                      ADmat)
    adst_pad = jnp.concatenate(
        [adst_tab, jnp.zeros((NSC - N, 16), f32)], axis=0)

    zacc = jnp.zeros((NSC, D), f32)
    zden = jnp.zeros((NSC, 16), f32)
    mvec = m8[2]
    acc_p, den_p = _edge(h, asrc_tab, adst_pad, mvec, sd, zacc, zden)

    bg2 = jnp.broadcast_to(bias_gat.astype(f32).reshape(1, D), (8, D))
    b22 = jnp.broadcast_to(b2.astype(f32).reshape(1, D), (8, D))
    out = _final(acc_p, den_p, h, asrc_tab, adst_tab, m8, EXPAND, bg2,
                 W2.astype(f32), b22)
    return out
